# baseline (speedup 1.0000x reference)
import functools

import jax
import jax.numpy as jnp
from jax import lax
from jax.experimental import pallas as pl
from jax.experimental.pallas import tpu as pltpu
from jax.experimental.pallas import tpu_sc as plsc

VOCAB = 100000
N_EMBD = 1024
CTX = 4096
BATCH = 4
SEQ = 4096

NUM_CORES = 2
NUM_SUBCORES = 16
NUM_WORKERS = NUM_CORES * NUM_SUBCORES
LANES = 16

B_TOT = BATCH * SEQ
ROWS_PER_W = B_TOT // NUM_WORKERS
CHUNK = 32
STEPS = ROWS_PER_W // CHUNK


def _embed_sc(ids_flat, token_table, pos_table):
    mesh = plsc.VectorSubcoreMesh(core_axis_name="c", subcore_axis_name="s")

    @functools.partial(
        pl.kernel,
        out_type=jax.ShapeDtypeStruct((B_TOT, N_EMBD), jnp.float32),
        mesh=mesh,
        scratch_types=[
            pltpu.VMEM((CHUNK,), jnp.int32),
            pltpu.VMEM((CHUNK, N_EMBD), jnp.float32),
            pltpu.VMEM((CHUNK, N_EMBD), jnp.float32),
            pltpu.SemaphoreType.DMA,
        ],
    )
    def k(ids_hbm, tok_hbm, pos_hbm, out_hbm, idx_v, rows_v, pos_v, sem):
        wid = lax.axis_index("s") * NUM_CORES + lax.axis_index("c")
        w_base = wid * ROWS_PER_W

        @pl.loop(0, STEPS)
        def _(j):
            base = w_base + j * CHUNK
            pos_base = lax.rem(base, SEQ)
            pltpu.sync_copy(ids_hbm.at[pl.ds(base, CHUNK)], idx_v)
            pltpu.async_copy(tok_hbm.at[idx_v], rows_v, sem).wait()
            pltpu.sync_copy(pos_hbm.at[pl.ds(pos_base, CHUNK)], pos_v)

            @pl.loop(0, CHUNK)
            def _(r):
                @pl.loop(0, N_EMBD // LANES)
                def _(c):
                    sl = pl.ds(c * LANES, LANES)
                    plsc.addupdate(rows_v.at[r, sl], pos_v[r, sl])

            pltpu.sync_copy(rows_v, out_hbm.at[pl.ds(base, CHUNK)])

    return k(ids_flat, token_table, pos_table)


@jax.jit
def kernel(token_ids, token_table, pos_table):
    ids_flat = token_ids.reshape(B_TOT)
    out = _embed_sc(ids_flat, token_table, pos_table)
    return out.reshape(BATCH, SEQ, N_EMBD)


# speedup vs baseline: 1.3605x; 1.3605x over previous
import functools

import jax
import jax.numpy as jnp
from jax import lax
from jax.experimental import pallas as pl
from jax.experimental.pallas import tpu as pltpu
from jax.experimental.pallas import tpu_sc as plsc

VOCAB = 100000
N_EMBD = 1024
CTX = 4096
BATCH = 4
SEQ = 4096

NUM_CORES = 2
NUM_SUBCORES = 16
NUM_WORKERS = NUM_CORES * NUM_SUBCORES
LANES = 16

POS_PER_W = SEQ // NUM_WORKERS
CHUNK = 32
PCHUNKS = POS_PER_W // CHUNK
UNITS = PCHUNKS * BATCH


def _embed_sc(ids_flat, token_table, pos_table):
    mesh = plsc.VectorSubcoreMesh(core_axis_name="c", subcore_axis_name="s")

    @functools.partial(
        pl.kernel,
        out_type=jax.ShapeDtypeStruct((BATCH * SEQ, N_EMBD), jnp.float32),
        mesh=mesh,
        scratch_types=[
            pltpu.VMEM((BATCH * POS_PER_W,), jnp.int32),
            pltpu.VMEM((CHUNK, N_EMBD), jnp.float32),
            pltpu.VMEM((CHUNK, N_EMBD), jnp.float32),
            pltpu.VMEM((CHUNK, N_EMBD), jnp.float32),
            pltpu.SemaphoreType.DMA,
            pltpu.SemaphoreType.DMA,
            pltpu.SemaphoreType.DMA,
            pltpu.SemaphoreType.DMA,
        ],
    )
    def k(ids_hbm, tok_hbm, pos_hbm, out_hbm,
          idx_v, pos_v, rows0, rows1, gsem0, gsem1, osem0, osem1):
        wid = lax.axis_index("s") * NUM_CORES + lax.axis_index("c")
        pbase = wid * POS_PER_W

        for b in range(BATCH):
            pltpu.sync_copy(ids_hbm.at[pl.ds(b * SEQ + pbase, POS_PER_W)],
                            idx_v.at[pl.ds(b * POS_PER_W, POS_PER_W)])
        pltpu.sync_copy(pos_hbm.at[pl.ds(pbase, CHUNK)], pos_v)

        rows = [rows0, rows1]
        gsem = [gsem0, gsem1]
        osem = [osem0, osem1]
        pending_g = [None, None]
        pending_o = [None, None]

        def start_gather(i):
            b, p = i % BATCH, i // BATCH
            buf = i % 2
            idx_sl = idx_v.at[pl.ds(b * POS_PER_W + p * CHUNK, CHUNK)]
            pending_g[buf] = pltpu.async_copy(
                tok_hbm.at[idx_sl], rows[buf], gsem[buf])

        start_gather(0)
        for i in range(UNITS):
            b, p = i % BATCH, i // BATCH
            buf = i % 2
            if i + 1 < UNITS:
                nbuf = (i + 1) % 2
                if pending_o[nbuf] is not None:
                    pending_o[nbuf].wait()
                    pending_o[nbuf] = None
                start_gather(i + 1)
            pending_g[buf].wait()

            rv = rows[buf]

            @pl.loop(0, CHUNK)
            def _(r, rv=rv):
                @pl.loop(0, N_EMBD // LANES, step=8)
                def _(c, r=r, rv=rv):
                    for u in range(8):
                        sl = pl.ds((c + u) * LANES, LANES)
                        plsc.addupdate(rv.at[r, sl], pos_v[r, sl])

            if b == BATCH - 1 and i + 1 < UNITS:
                pltpu.sync_copy(
                    pos_hbm.at[pl.ds(pbase + (p + 1) * CHUNK, CHUNK)], pos_v)

            pending_o[buf] = pltpu.async_copy(
                rows[buf],
                out_hbm.at[pl.ds(b * SEQ + pbase + p * CHUNK, CHUNK)],
                osem[buf])

        pending_o[0].wait()
        pending_o[1].wait()

    return k(ids_flat, token_table, pos_table)


@jax.jit
def kernel(token_ids, token_table, pos_table):
    ids_flat = token_ids.reshape(BATCH * SEQ)
    out = _embed_sc(ids_flat, token_table, pos_table)
    return out.reshape(BATCH, SEQ, N_EMBD)


# speedup vs baseline: 2.8084x; 2.0642x over previous
import functools

import jax
import jax.numpy as jnp
from jax import lax
from jax.experimental import pallas as pl
from jax.experimental.pallas import tpu as pltpu
from jax.experimental.pallas import tpu_sc as plsc

VOCAB = 100000
N_EMBD = 1024
CTX = 4096
BATCH = 4
SEQ = 4096

NUM_CORES = 2
NUM_SUBCORES = 16
NUM_WORKERS = NUM_CORES * NUM_SUBCORES
LANES = 16

POS_PER_W = SEQ // NUM_WORKERS
CHUNK = 32
PCHUNKS = POS_PER_W // CHUNK
UNITS = PCHUNKS * BATCH


def _embed_sc(ids_flat, token_table, pos_table):
    mesh = plsc.VectorSubcoreMesh(core_axis_name="c", subcore_axis_name="s")

    @functools.partial(
        pl.kernel,
        out_type=jax.ShapeDtypeStruct((BATCH * SEQ, N_EMBD), jnp.float32),
        mesh=mesh,
        scratch_types=[
            pltpu.VMEM((BATCH * POS_PER_W,), jnp.int32),
            pltpu.VMEM((CHUNK, N_EMBD), jnp.float32),
            pltpu.VMEM((CHUNK, N_EMBD), jnp.float32),
            pltpu.VMEM((CHUNK, N_EMBD), jnp.float32),
            pltpu.SemaphoreType.DMA,
            pltpu.SemaphoreType.DMA,
            pltpu.SemaphoreType.DMA,
            pltpu.SemaphoreType.DMA,
        ],
    )
    def k(ids_hbm, tok_hbm, pos_hbm, out_hbm,
          idx_v, pos_v, rows0, rows1, gsem0, gsem1, osem0, osem1):
        wid = lax.axis_index("s") * NUM_CORES + lax.axis_index("c")
        pbase = wid * POS_PER_W

        for b in range(BATCH):
            pltpu.sync_copy(ids_hbm.at[pl.ds(b * SEQ + pbase, POS_PER_W)],
                            idx_v.at[pl.ds(b * POS_PER_W, POS_PER_W)])
        pltpu.sync_copy(pos_hbm.at[pl.ds(pbase, CHUNK)], pos_v)

        rows = [rows0, rows1]
        gsem = [gsem0, gsem1]
        osem = [osem0, osem1]
        pending_g = [None, None]
        pending_o = [None, None]

        def start_gather(i):
            b, p = i % BATCH, i // BATCH
            buf = i % 2
            idx_sl = idx_v.at[pl.ds(b * POS_PER_W + p * CHUNK, CHUNK)]
            pending_g[buf] = pltpu.async_copy(
                tok_hbm.at[idx_sl], rows[buf], gsem[buf])

        start_gather(0)
        for i in range(UNITS):
            b, p = i % BATCH, i // BATCH
            buf = i % 2
            if i + 1 < UNITS:
                nbuf = (i + 1) % 2
                if pending_o[nbuf] is not None:
                    pending_o[nbuf].wait()
                    pending_o[nbuf] = None
                start_gather(i + 1)
            pending_g[buf].wait()

            rv = rows[buf]

            @plsc.parallel_loop(0, CHUNK * N_EMBD // LANES, unroll=8)
            def _(t, rv=rv):
                r = t >> 6
                sl = pl.ds((t & (N_EMBD // LANES - 1)) * LANES, LANES)
                plsc.addupdate(rv.at[r, sl], pos_v[r, sl])

            if b == BATCH - 1 and i + 1 < UNITS:
                pltpu.sync_copy(
                    pos_hbm.at[pl.ds(pbase + (p + 1) * CHUNK, CHUNK)], pos_v)

            pending_o[buf] = pltpu.async_copy(
                rows[buf],
                out_hbm.at[pl.ds(b * SEQ + pbase + p * CHUNK, CHUNK)],
                osem[buf])

        pending_o[0].wait()
        pending_o[1].wait()

    return k(ids_flat, token_table, pos_table)


@jax.jit
def kernel(token_ids, token_table, pos_table):
    ids_flat = token_ids.reshape(BATCH * SEQ)
    out = _embed_sc(ids_flat, token_table, pos_table)
    return out.reshape(BATCH, SEQ, N_EMBD)
